# Initial kernel scaffold; baseline (speedup 1.0000x reference)
import jax
import jax.numpy as jnp
from jax.experimental import pallas as pl


def kernel(X, H, C, M, T):
    raise NotImplementedError("write your pallas kernel here")


# speedup vs baseline: 7.4900x; 7.4900x over previous
import functools

import jax
import jax.numpy as jnp
from jax.experimental import pallas as pl

ROW_PENALTY = 10.0


def _loss_kernel(x_ref, h_ref, c_ref, m_ref, out_ref):
    X = x_ref[...]
    H = h_ref[...]
    C = c_ref[...]
    M = m_ref[...]
    T = X.shape[0]

    f32 = jnp.float32
    mask = (M > 0).astype(f32)
    U = mask * H
    V = U * H

    dot_t = functools.partial(
        jax.lax.dot_general,
        dimension_numbers=(((1,), (1,)), ((), ())),
        preferred_element_type=f32,
        precision=jax.lax.Precision.HIGHEST,
    )

    cnt = dot_t(mask, mask)
    B = dot_t(U, mask)
    P = dot_t(V, mask)
    Q = dot_t(U, U)

    S1 = B - B.T
    S2 = P + P.T - 2.0 * Q
    mean = S1 / jnp.maximum(cnt, 1.0)
    var_num = S2 - mean * (2.0 * S1 - cnt * mean)
    var = var_num / jnp.maximum(cnt - 1.0, 1.0)

    d_iota = jax.lax.broadcasted_iota(jnp.int32, M.shape, 1)
    row_min = jnp.min(M, axis=1, keepdims=True)
    am = jnp.min(jnp.where(M == row_min, d_iota, M.shape[1]), axis=1,
                 keepdims=True)

    iota_r = jax.lax.broadcasted_iota(jnp.int32, (T, T), 0)
    iota_c = jax.lax.broadcasted_iota(jnp.int32, (T, T), 1)
    valid = (iota_r != iota_c) & (am != am.reshape(1, T))
    scores = jnp.where(valid, var, 9999.0)

    s_min = jnp.min(scores, axis=1, keepdims=True)
    min_row = jnp.min(jnp.where(scores == s_min, iota_c, T), axis=1,
                      keepdims=True)

    onehot = (min_row == iota_c).astype(f32)
    G = jax.lax.dot_general(
        onehot, H, dimension_numbers=(((1,), (0,)), ((), ())),
        preferred_element_type=f32,
        precision=jax.lax.Precision.HIGHEST,
    )
    diff = H - G
    row_loss = jnp.sum(jnp.sqrt(jnp.sum(diff * diff, axis=1)))

    resid = X * M - (H - C) * M
    mse = jnp.sum(resid * resid)

    out_ref[...] = jnp.reshape(mse + ROW_PENALTY * row_loss, (1, 1))


def kernel(X, H, C, M, T):
    out = pl.pallas_call(
        _loss_kernel,
        out_shape=jax.ShapeDtypeStruct((1, 1), jnp.float32),
    )(X, H, C, M)
    return out[0, 0]


# speedup vs baseline: 10.4760x; 1.3987x over previous
import functools

import jax
import jax.numpy as jnp
from jax.experimental import pallas as pl

ROW_PENALTY = 10.0


def _loss_kernel(x_ref, h_ref, c_ref, m_ref, out_ref):
    X = x_ref[...]
    H = h_ref[...]
    C = c_ref[...]
    M = m_ref[...]
    T = X.shape[0]

    f32 = jnp.float32
    mask = (M > 0).astype(f32)
    U = mask * H
    V = U * H

    dot_t = functools.partial(
        jax.lax.dot_general,
        dimension_numbers=(((1,), (1,)), ((), ())),
        preferred_element_type=f32,
    )

    cnt = dot_t(mask, mask)
    B = dot_t(U, mask)
    P = dot_t(V, mask)
    Q = dot_t(U, U)
    R = dot_t(H, H)

    S1 = B - B.T
    S2 = P + P.T - 2.0 * Q
    mean = S1 / jnp.maximum(cnt, 1.0)
    var_num = S2 - mean * (2.0 * S1 - cnt * mean)
    var = var_num / jnp.maximum(cnt - 1.0, 1.0)

    d_iota = jax.lax.broadcasted_iota(jnp.int32, M.shape, 1)
    row_min = jnp.min(M, axis=1, keepdims=True)
    am = jnp.min(jnp.where(M == row_min, d_iota, M.shape[1]), axis=1,
                 keepdims=True)

    iota_r = jax.lax.broadcasted_iota(jnp.int32, (T, T), 0)
    iota_c = jax.lax.broadcasted_iota(jnp.int32, (T, T), 1)
    valid = (iota_r != iota_c) & (am != am.reshape(1, T))
    scores = jnp.where(valid, var, 9999.0)

    s_min = jnp.min(scores, axis=1, keepdims=True)
    min_row = jnp.min(jnp.where(scores == s_min, iota_c, T), axis=1,
                      keepdims=True)
    onehot = min_row == iota_c

    h2 = jnp.sum(H * H, axis=1, keepdims=True)
    norm2 = jnp.maximum(h2 + h2.reshape(1, T) - 2.0 * R, 0.0)
    sel = jnp.sum(jnp.where(onehot, norm2, 0.0), axis=1)
    row_loss = jnp.sum(jnp.sqrt(sel))

    resid = X * M - (H - C) * M
    mse = jnp.sum(resid * resid)

    out_ref[...] = jnp.reshape(mse + ROW_PENALTY * row_loss, (1, 1))


def kernel(X, H, C, M, T):
    out = pl.pallas_call(
        _loss_kernel,
        out_shape=jax.ShapeDtypeStruct((1, 1), jnp.float32),
    )(X, H, C, M)
    return out[0, 0]
